# baseline (device time: 12825 ns/iter reference)
import jax
import jax.numpy as jnp
from jax import lax
from jax.experimental import pallas as pl
from jax.experimental.pallas import tpu as pltpu

N_DEV = 4
N_CHUNK = 2
HOP_ORDER = (2, 1, 3)


def kernel(x, Wg, Wu, Wd):
    m, _ = x.shape
    d = Wd.shape[1]
    mc = m // N_CHUNK

    def body(x_ref, wg_ref, wu_ref, wd_ref, out_ref,
             send_ref, comm_ref, send_sems, recv_sems):
        my_pos = lax.axis_index("i")

        barrier_sem = pltpu.get_barrier_semaphore()
        for h in range(1, N_DEV):
            pl.semaphore_signal(
                barrier_sem, inc=1,
                device_id=(lax.rem(my_pos + h, N_DEV),),
                device_id_type=pl.DeviceIdType.MESH,
            )

        xb = x_ref[...].astype(jnp.bfloat16)
        gate = jnp.dot(xb, wg_ref[...].astype(jnp.bfloat16),
                       preferred_element_type=jnp.float32)
        up = jnp.dot(xb, wu_ref[...].astype(jnp.bfloat16),
                     preferred_element_type=jnp.float32)
        hidden = (gate * (up * jax.nn.sigmoid(up))).astype(jnp.bfloat16)
        wdb = wd_ref[...].astype(jnp.bfloat16)

        partials = []
        rdmas = []
        for c in range(N_CHUNK):
            p = jnp.dot(hidden[c * mc:(c + 1) * mc, :], wdb,
                        preferred_element_type=jnp.float32)
            partials.append(p)
            send_ref[c] = p.astype(jnp.bfloat16)
            if c == 0:
                pl.semaphore_wait(barrier_sem, N_DEV - 1)
            for h in HOP_ORDER:
                rdma = pltpu.make_async_remote_copy(
                    src_ref=send_ref.at[c],
                    dst_ref=comm_ref.at[h - 1, c],
                    send_sem=send_sems.at[h - 1, c],
                    recv_sem=recv_sems.at[h - 1, c],
                    device_id=(lax.rem(my_pos + h, N_DEV),),
                    device_id_type=pl.DeviceIdType.MESH,
                )
                rdma.start()
                rdmas.append(rdma)

        for c in range(N_CHUNK):
            for i in range(len(HOP_ORDER)):
                rdmas[c * len(HOP_ORDER) + i].wait_recv()
            out_ref[c * mc:(c + 1) * mc, :] = (
                partials[c]
                + comm_ref[0, c].astype(jnp.float32)
                + comm_ref[1, c].astype(jnp.float32)
                + comm_ref[2, c].astype(jnp.float32))

        for rdma in rdmas:
            rdma.wait_send()

    return pl.pallas_call(
        body,
        out_shape=jax.ShapeDtypeStruct((m, d), jnp.float32),
        in_specs=[pl.BlockSpec(memory_space=pltpu.VMEM)] * 4,
        out_specs=pl.BlockSpec(memory_space=pltpu.VMEM),
        scratch_shapes=[
            pltpu.VMEM((N_CHUNK, mc, d), jnp.bfloat16),
            pltpu.VMEM((N_DEV - 1, N_CHUNK, mc, d), jnp.bfloat16),
            pltpu.SemaphoreType.DMA((N_DEV - 1, N_CHUNK)),
            pltpu.SemaphoreType.DMA((N_DEV - 1, N_CHUNK)),
        ],
        compiler_params=pltpu.CompilerParams(collective_id=0),
    )(x, Wg, Wu, Wd)


# device time: 12804 ns/iter; 1.0016x vs baseline; 1.0016x over previous
import os

import jax
import jax.numpy as jnp
from jax import lax
from jax.experimental import pallas as pl
from jax.experimental.pallas import tpu as pltpu

try:
    VARIANT = (
        open(os.path.join(os.path.dirname(__file__), "variant.txt")).read().strip()
        or "full"
    )
except OSError:
    VARIANT = "full"

N_DEV = 4
N_CHUNK = 2
HOP_ORDER = (2, 1, 3)


def kernel(x, Wg, Wu, Wd):
    m, _ = x.shape
    d = Wd.shape[1]
    mc = m // N_CHUNK

    def body(x_ref, wg_ref, wu_ref, wd_ref, out_ref,
             send_ref, comm_ref, send_sems, recv_sems):
        my_pos = lax.axis_index("i")

        barrier_sem = pltpu.get_barrier_semaphore()
        for h in range(1, N_DEV):
            pl.semaphore_signal(
                barrier_sem, inc=1,
                device_id=(lax.rem(my_pos + h, N_DEV),),
                device_id_type=pl.DeviceIdType.MESH,
            )

        xb = x_ref[...].astype(jnp.bfloat16)
        gate = jnp.dot(xb, wg_ref[...].astype(jnp.bfloat16),
                       preferred_element_type=jnp.float32)
        up = jnp.dot(xb, wu_ref[...].astype(jnp.bfloat16),
                     preferred_element_type=jnp.float32)
        hidden = (gate * (up * jax.nn.sigmoid(up))).astype(jnp.bfloat16)
        wdb = wd_ref[...].astype(jnp.bfloat16)

        partials = []
        rdmas = []
        for c in range(N_CHUNK):
            p = jnp.dot(hidden[c * mc:(c + 1) * mc, :], wdb,
                        preferred_element_type=jnp.float32)
            partials.append(p)
            send_ref[c] = p.astype(jnp.bfloat16)
            if c == 0:
                pl.semaphore_wait(barrier_sem, N_DEV - 1)
            if VARIANT == "barrier":
                continue
            for h in HOP_ORDER:
                rdma = pltpu.make_async_remote_copy(
                    src_ref=send_ref.at[c],
                    dst_ref=comm_ref.at[h - 1, c],
                    send_sem=send_sems.at[h - 1, c],
                    recv_sem=recv_sems.at[h - 1, c],
                    device_id=(lax.rem(my_pos + h, N_DEV),),
                    device_id_type=pl.DeviceIdType.MESH,
                )
                rdma.start()
                rdmas.append(rdma)

        for c in range(N_CHUNK):
            if VARIANT != "barrier":
                for i in range(len(HOP_ORDER)):
                    rdmas[c * len(HOP_ORDER) + i].wait_recv()
            if VARIANT in ("nosum", "barrier"):
                out_ref[c * mc:(c + 1) * mc, :] = partials[c]
            else:
                out_ref[c * mc:(c + 1) * mc, :] = (
                    partials[c]
                    + comm_ref[0, c].astype(jnp.float32)
                    + comm_ref[1, c].astype(jnp.float32)
                    + comm_ref[2, c].astype(jnp.float32))

        for rdma in rdmas:
            rdma.wait_send()

    return pl.pallas_call(
        body,
        out_shape=jax.ShapeDtypeStruct((m, d), jnp.float32),
        in_specs=[pl.BlockSpec(memory_space=pltpu.VMEM)] * 4,
        out_specs=pl.BlockSpec(memory_space=pltpu.VMEM),
        scratch_shapes=[
            pltpu.VMEM((N_CHUNK, mc, d), jnp.bfloat16),
            pltpu.VMEM((N_DEV - 1, N_CHUNK, mc, d), jnp.bfloat16),
            pltpu.SemaphoreType.DMA((N_DEV - 1, N_CHUNK)),
            pltpu.SemaphoreType.DMA((N_DEV - 1, N_CHUNK)),
        ],
        compiler_params=pltpu.CompilerParams(collective_id=0),
    )(x, Wg, Wu, Wd)
